# Initial kernel scaffold; baseline (speedup 1.0000x reference)
import jax
import jax.numpy as jnp
from jax.experimental import pallas as pl


def kernel(x, edge_index, batch, W1, b1, W2, b2, W3, b3, Wp, bp, Wl1, bl1, Wl2, bl2, Wl3, bl3):
    raise NotImplementedError("write your pallas kernel here")


# speedup vs baseline: 7.4400x; 7.4400x over previous
import functools

import jax
import jax.numpy as jnp
from jax import lax
from jax.experimental import pallas as pl
from jax.experimental.pallas import tpu as pltpu
from jax.experimental.pallas import tpu_sc as plsc

N = 10000
E = 320000
G = 64
NHID = 128
NP = 10240
RPT = NP // 16
NW = 32
C = 128
NCH = 80
EPAD = NW * NCH * C
DUMMY = 10104


def _make_sc_agg(D):
  mesh = plsc.VectorSubcoreMesh(core_axis_name="c", subcore_axis_name="s")

  @functools.partial(
      pl.kernel,
      mesh=mesh,
      out_type=jax.ShapeDtypeStruct((2, NP, D), jnp.float32),
      scratch_types=[
          pltpu.VMEM((NCH, C), jnp.int32),
          pltpu.VMEM((NCH, C), jnp.int32),
          pltpu.VMEM((C, D), jnp.float32),
          pltpu.VMEM_SHARED((NP, D), jnp.float32),
          pltpu.SemaphoreType.DMA,
      ],
  )
  def agg(hs_hbm, rowi_hbm, coli_hbm, out_hbm, rowv, colv, buf, acc, sem):
    c = lax.axis_index("c")
    s = lax.axis_index("s")
    wid = s * 2 + c
    r0 = s * RPT
    pltpu.sync_copy(hs_hbm.at[pl.ds(r0, RPT)], acc.at[pl.ds(r0, RPT)])
    pltpu.sync_copy(rowi_hbm.at[wid], rowv)
    pltpu.sync_copy(coli_hbm.at[wid], colv)
    plsc.subcore_barrier()

    def body(j, carry):
      pltpu.async_copy(hs_hbm.at[rowv.at[j]], buf, sem).wait()
      pltpu.sync_copy(buf, acc.at[colv.at[j]], add=True)
      return carry

    lax.fori_loop(0, NCH, body, 0)
    plsc.subcore_barrier()
    pltpu.sync_copy(acc.at[pl.ds(r0, RPT)], out_hbm.at[c].at[pl.ds(r0, RPT)])

  return agg


_sc_agg128 = _make_sc_agg(NHID)


def _make_sc_agg1():
  mesh = plsc.VectorSubcoreMesh(core_axis_name="c", subcore_axis_name="s")

  @functools.partial(
      pl.kernel,
      mesh=mesh,
      out_type=jax.ShapeDtypeStruct((2, NP), jnp.float32),
      scratch_types=[
          pltpu.VMEM((NCH, C), jnp.int32),
          pltpu.VMEM((NCH, C), jnp.int32),
          pltpu.VMEM((C,), jnp.float32),
          pltpu.VMEM_SHARED((NP,), jnp.float32),
          pltpu.SemaphoreType.DMA,
      ],
  )
  def agg(hs_hbm, rowi_hbm, coli_hbm, out_hbm, rowv, colv, buf, acc, sem):
    c = lax.axis_index("c")
    s = lax.axis_index("s")
    wid = s * 2 + c
    r0 = s * RPT
    pltpu.sync_copy(hs_hbm.at[pl.ds(r0, RPT)], acc.at[pl.ds(r0, RPT)])
    pltpu.sync_copy(rowi_hbm.at[wid], rowv)
    pltpu.sync_copy(coli_hbm.at[wid], colv)
    plsc.subcore_barrier()

    def body(j, carry):
      pltpu.async_copy(hs_hbm.at[rowv.at[j]], buf, sem).wait()
      pltpu.sync_copy(buf, acc.at[colv.at[j]], add=True)
      return carry

    lax.fori_loop(0, NCH, body, 0)
    plsc.subcore_barrier()
    pltpu.sync_copy(acc.at[pl.ds(r0, RPT)], out_hbm.at[c].at[pl.ds(r0, RPT)])

  return agg


_sc_agg1 = _make_sc_agg1()


def _prep_body(x_ref, w1_ref, degp_ref, hs1_ref, dinv_ref):
  deg = degp_ref[0, :, 0:1] + degp_ref[1, :, 0:1] - 1.0
  dinv = lax.rsqrt(deg)
  dinv_ref[...] = dinv
  hs1_ref[...] = jnp.dot(x_ref[...], w1_ref[...],
                         preferred_element_type=jnp.float32) * dinv


def _tc_prep(x, w1, degp):
  return pl.pallas_call(
      _prep_body,
      out_shape=[
          jax.ShapeDtypeStruct((NP, NHID), jnp.float32),
          jax.ShapeDtypeStruct((NP, 1), jnp.float32),
      ],
  )(x, w1, degp)


def _layer_body(p_ref, hs_ref, dinv_ref, b_ref, wn_ref, xl_ref, hsn_ref):
  dinv = dinv_ref[...]
  xl = jnp.maximum(dinv * (p_ref[0] + p_ref[1] - hs_ref[...]) + b_ref[...],
                   0.0)
  xl_ref[...] = xl
  hsn_ref[...] = jnp.dot(xl, wn_ref[...],
                         preferred_element_type=jnp.float32) * dinv


def _tc_layer(p, hs, dinv, b, wn):
  return pl.pallas_call(
      _layer_body,
      out_shape=[
          jax.ShapeDtypeStruct((NP, NHID), jnp.float32),
          jax.ShapeDtypeStruct((NP, NHID), jnp.float32),
      ],
  )(p, hs, dinv, b, wn)


def _layer3_body(p_ref, hs_ref, dinv_ref, b_ref, x1_ref, x2_ref, wp_ref,
                 x3_ref, ssw_ref):
  dinv = dinv_ref[...]
  x3 = jnp.maximum(dinv * (p_ref[0] + p_ref[1] - hs_ref[...]) + b_ref[...],
                   0.0)
  x3_ref[...] = x3
  wp = wp_ref[...]
  s = (jnp.dot(x1_ref[...], wp[0:128], preferred_element_type=jnp.float32)
       + jnp.dot(x2_ref[...], wp[128:256], preferred_element_type=jnp.float32)
       + jnp.dot(x3, wp[256:384], preferred_element_type=jnp.float32))
  ssw_ref[...] = s * dinv


def _tc_layer3(p, hs, dinv, b, x1, x2, wp):
  return pl.pallas_call(
      _layer3_body,
      out_shape=[
          jax.ShapeDtypeStruct((NP, NHID), jnp.float32),
          jax.ShapeDtypeStruct((NP, 1), jnp.float32),
      ],
  )(p, hs, dinv, b, x1, x2, wp)


def _poolA_body(sp_ref, ssw_ref, dinv_ref, bp_ref, batch_ref,
                wk_ref, keep_ref, kcol_ref):
  score = (dinv_ref[...]
           * (sp_ref[0] + sp_ref[1] - ssw_ref[...])
           + bp_ref[0, 0])
  batch = batch_ref[...]
  gids = lax.broadcasted_iota(jnp.int32, (1, G), 1)
  oh = (batch == gids)
  counts = jnp.sum(oh.astype(jnp.float32), axis=0, keepdims=True)
  kf = jnp.ceil(0.5 * counts)

  u = lax.bitcast_convert_type(score, jnp.uint32)
  flip = jnp.where(u >= jnp.uint32(0x80000000),
                   jnp.uint32(0xFFFFFFFF), jnp.uint32(0x80000000))
  key = u ^ flip

  def bs_body(i, t):
    b = (jnp.uint32(31) - i.astype(jnp.uint32))
    cand = t | (jnp.uint32(1) << b)
    ge = (key >= cand) & oh
    cnt = jnp.sum(jnp.where(ge, 1.0, 0.0), axis=0, keepdims=True)
    return jnp.where(cnt >= kf, cand, t)

  t = lax.fori_loop(0, 32, bs_body, jnp.zeros((1, G), jnp.uint32))
  ti = lax.bitcast_convert_type(t, jnp.int32)
  tpn = lax.bitcast_convert_type(
      jnp.sum(jnp.where(oh, ti, 0), axis=1, keepdims=True), jnp.uint32)
  keep = key >= tpn
  w = jnp.tanh(score)
  wk_ref[...] = jnp.where(keep, w, 0.0)
  keep_ref[...] = jnp.where(keep, 1.0, 0.0)
  kcol_ref[...] = kf.reshape(G, 1)


def _tc_poolA(sp, ssw, dinv, bp, batch2):
  return pl.pallas_call(
      _poolA_body,
      out_shape=[
          jax.ShapeDtypeStruct((NP, 1), jnp.float32),
          jax.ShapeDtypeStruct((NP, 1), jnp.float32),
          jax.ShapeDtypeStruct((G, 1), jnp.float32),
      ],
  )(sp, ssw, dinv, bp, batch2)


def _poolB_body(x1_ref, x2_ref, x3_ref, wk_ref, keep_ref, kcol_ref,
                batch_ref, batchr_ref, wl1_ref, bl1_ref, wl2_ref, bl2_ref,
                wl3_ref, bl3_ref, out_ref, mx1, mx2, mx3):
  wk = wk_ref[...]
  v1 = x1_ref[...] * wk
  v2 = x2_ref[...] * wk
  v3 = x3_ref[...] * wk
  ohT = (lax.broadcasted_iota(jnp.int32, (G, 1), 0)
         == batchr_ref[...]).astype(jnp.float32)
  ssum1 = jnp.dot(ohT, v1, preferred_element_type=jnp.float32)
  ssum2 = jnp.dot(ohT, v2, preferred_element_type=jnp.float32)
  ssum3 = jnp.dot(ohT, v3, preferred_element_type=jnp.float32)

  neg = jnp.float32(-jnp.inf)
  batch = batch_ref[...]
  keep = keep_ref[...] > 0.0

  def mx_body(g, carry):
    m = (batch == g) & keep
    m1 = jnp.max(jnp.where(m, v1, neg), axis=0, keepdims=True)
    m2 = jnp.max(jnp.where(m, v2, neg), axis=0, keepdims=True)
    m3 = jnp.max(jnp.where(m, v3, neg), axis=0, keepdims=True)
    mx1[pl.ds(g, 1), :] = m1
    mx2[pl.ds(g, 1), :] = m2
    mx3[pl.ds(g, 1), :] = m3
    return carry

  lax.fori_loop(0, G, mx_body, 0)

  kcol = kcol_ref[...]
  inv_k = 1.0 / jnp.maximum(kcol, 1.0)
  gap1, gap2, gap3 = ssum1 * inv_k, ssum2 * inv_k, ssum3 * inv_k
  nz = kcol > 0
  gmp1 = jnp.where(nz, mx1[...], 0.0)
  gmp2 = jnp.where(nz, mx2[...], 0.0)
  gmp3 = jnp.where(nz, mx3[...], 0.0)

  wl1 = wl1_ref[...]
  z = (jnp.dot(gmp1, wl1[0:128], preferred_element_type=jnp.float32)
       + jnp.dot(gmp2, wl1[128:256], preferred_element_type=jnp.float32)
       + jnp.dot(gmp3, wl1[256:384], preferred_element_type=jnp.float32)
       + jnp.dot(gap1, wl1[384:512], preferred_element_type=jnp.float32)
       + jnp.dot(gap2, wl1[512:640], preferred_element_type=jnp.float32)
       + jnp.dot(gap3, wl1[640:768], preferred_element_type=jnp.float32)
       + bl1_ref[...])
  z = jnp.maximum(z, 0.0)
  z = jnp.maximum(jnp.dot(z, wl2_ref[...],
                          preferred_element_type=jnp.float32) + bl2_ref[...],
                  0.0)
  z = jnp.dot(z, wl3_ref[...], preferred_element_type=jnp.float32) + bl3_ref[...]
  m = jnp.max(z, axis=1, keepdims=True)
  lse = jnp.log(jnp.sum(jnp.exp(z - m), axis=1, keepdims=True))
  out_ref[...] = z - m - lse


def _tc_poolB(x1, x2, x3, wk, keepf, kcol, batch2, batchr, wl1, bl1, wl2,
              bl2, wl3, bl3):
  return pl.pallas_call(
      _poolB_body,
      out_shape=jax.ShapeDtypeStruct((G, 10), jnp.float32),
      scratch_shapes=[
          pltpu.VMEM((G, NHID), jnp.float32),
          pltpu.VMEM((G, NHID), jnp.float32),
          pltpu.VMEM((G, NHID), jnp.float32),
      ],
  )(x1, x2, x3, wk, keepf, kcol, batch2, batchr, wl1, bl1, wl2, bl2,
    wl3, bl3)


@jax.jit
def kernel(x, edge_index, batch, W1, b1, W2, b2, W3, b3, Wp, bp,
           Wl1, bl1, Wl2, bl2, Wl3, bl3):
  xp = jnp.pad(x, ((0, NP - N), (0, 0)))
  row = jnp.pad(edge_index[0], (0, EPAD - E), constant_values=DUMMY)
  col = jnp.pad(edge_index[1], (0, EPAD - E), constant_values=DUMMY)
  rowi = row.reshape(NW, NCH, C)
  coli = col.reshape(NW, NCH, C)
  batch_p = jnp.pad(batch, (0, NP - N), constant_values=G)
  batch2 = batch_p.reshape(NP, 1)
  batchr = batch_p.reshape(1, NP)
  ones1 = jnp.ones((NP,), jnp.float32)

  degp = _sc_agg1(ones1, rowi, coli).reshape(2, NP, 1)
  hs1, dinv = _tc_prep(xp, W1, degp)
  p1 = _sc_agg128(hs1, rowi, coli)
  x1, hs2 = _tc_layer(p1, hs1, dinv, b1.reshape(1, NHID), W2)
  p2 = _sc_agg128(hs2, rowi, coli)
  x2, hs3 = _tc_layer(p2, hs2, dinv, b2.reshape(1, NHID), W3)
  p3 = _sc_agg128(hs3, rowi, coli)
  x3, ssw = _tc_layer3(p3, hs3, dinv, b3.reshape(1, NHID), x1, x2, Wp)
  sp = _sc_agg1(ssw.reshape(NP), rowi, coli).reshape(2, NP, 1)
  wk, keepf, kcol = _tc_poolA(sp, ssw, dinv, bp.reshape(1, 1), batch2)
  return _tc_poolB(x1, x2, x3, wk, keepf, kcol, batch2, batchr, Wl1,
                   bl1.reshape(1, 2 * NHID), Wl2, bl2.reshape(1, NHID),
                   Wl3, bl3.reshape(1, 10))


# speedup vs baseline: 7.5391x; 1.0133x over previous
import functools

import jax
import jax.numpy as jnp
from jax import lax
from jax.experimental import pallas as pl
from jax.experimental.pallas import tpu as pltpu
from jax.experimental.pallas import tpu_sc as plsc

N = 10000
E = 320000
G = 64
NHID = 128
NP = 10240
RPT = NP // 16
NW = 32
C = 128
NCH = 80
EPAD = NW * NCH * C
DUMMY = 10104


def _make_sc_agg(D):
  mesh = plsc.VectorSubcoreMesh(core_axis_name="c", subcore_axis_name="s")

  @functools.partial(
      pl.kernel,
      mesh=mesh,
      out_type=jax.ShapeDtypeStruct((2, NP, D), jnp.float32),
      scratch_types=[
          pltpu.VMEM((2, C), jnp.int32),
          pltpu.VMEM((2, C), jnp.int32),
          pltpu.VMEM((C, D), jnp.float32),
          pltpu.VMEM((C, D), jnp.float32),
          pltpu.VMEM_SHARED((NP, D), jnp.float32),
          pltpu.SemaphoreType.DMA,
          pltpu.SemaphoreType.DMA,
          pltpu.SemaphoreType.DMA,
          pltpu.SemaphoreType.DMA,
      ],
  )
  def agg(hs_hbm, eidx_hbm, out_hbm, idx0, idx1, buf0, buf1, acc,
          semi0, semi1, sem0, sem1):
    c = lax.axis_index("c")
    s = lax.axis_index("s")
    wid = s * 2 + c
    r0 = s * RPT
    pltpu.sync_copy(hs_hbm.at[pl.ds(r0, RPT)], acc.at[pl.ds(r0, RPT)])
    plsc.subcore_barrier()

    pltpu.sync_copy(eidx_hbm.at[wid, 0], idx0)
    pltpu.async_copy(hs_hbm.at[idx0.at[0]], buf0, sem0)
    pltpu.async_copy(eidx_hbm.at[wid, 1], idx1, semi1)

    def body(jj, carry):
      j0 = 2 * jj
      pltpu.make_async_copy(eidx_hbm.at[wid, j0 + 1], idx1, semi1).wait()
      pltpu.async_copy(hs_hbm.at[idx1.at[0]], buf1, sem1)
      pltpu.make_async_copy(hs_hbm.at[idx0.at[0]], buf0, sem0).wait()
      pltpu.sync_copy(buf0, acc.at[idx0.at[1]], add=True)

      @pl.when(jj + 1 < NCH // 2)
      def _():
        pltpu.async_copy(eidx_hbm.at[wid, j0 + 2], idx0, semi0)
        pltpu.make_async_copy(eidx_hbm.at[wid, j0 + 2], idx0, semi0).wait()
        pltpu.async_copy(hs_hbm.at[idx0.at[0]], buf0, sem0)

      pltpu.make_async_copy(hs_hbm.at[idx1.at[0]], buf1, sem1).wait()
      pltpu.sync_copy(buf1, acc.at[idx1.at[1]], add=True)

      @pl.when(jj + 1 < NCH // 2)
      def _():
        pltpu.async_copy(eidx_hbm.at[wid, j0 + 3], idx1, semi1)

      return carry

    lax.fori_loop(0, NCH // 2, body, 0)
    plsc.subcore_barrier()
    pltpu.sync_copy(acc.at[pl.ds(r0, RPT)], out_hbm.at[c].at[pl.ds(r0, RPT)])

  return agg


_sc_agg128 = _make_sc_agg(NHID)


def _make_sc_agg1():
  mesh = plsc.VectorSubcoreMesh(core_axis_name="c", subcore_axis_name="s")

  @functools.partial(
      pl.kernel,
      mesh=mesh,
      out_type=jax.ShapeDtypeStruct((2, NP), jnp.float32),
      scratch_types=[
          pltpu.VMEM((NCH, C), jnp.int32),
          pltpu.VMEM((NCH, C), jnp.int32),
          pltpu.VMEM((C,), jnp.float32),
          pltpu.VMEM_SHARED((NP,), jnp.float32),
          pltpu.SemaphoreType.DMA,
      ],
  )
  def agg(hs_hbm, rowi_hbm, coli_hbm, out_hbm, rowv, colv, buf, acc, sem):
    c = lax.axis_index("c")
    s = lax.axis_index("s")
    wid = s * 2 + c
    r0 = s * RPT
    pltpu.sync_copy(hs_hbm.at[pl.ds(r0, RPT)], acc.at[pl.ds(r0, RPT)])
    pltpu.sync_copy(rowi_hbm.at[wid], rowv)
    pltpu.sync_copy(coli_hbm.at[wid], colv)
    plsc.subcore_barrier()

    def body(j, carry):
      pltpu.async_copy(hs_hbm.at[rowv.at[j]], buf, sem).wait()
      pltpu.sync_copy(buf, acc.at[colv.at[j]], add=True)
      return carry

    lax.fori_loop(0, NCH, body, 0)
    plsc.subcore_barrier()
    pltpu.sync_copy(acc.at[pl.ds(r0, RPT)], out_hbm.at[c].at[pl.ds(r0, RPT)])

  return agg


_sc_agg1 = _make_sc_agg1()


def _prep_body(x_ref, w1_ref, degp_ref, hs1_ref, dinv_ref):
  deg = degp_ref[0, :, 0:1] + degp_ref[1, :, 0:1] - 1.0
  dinv = lax.rsqrt(deg)
  dinv_ref[...] = dinv
  hs1_ref[...] = jnp.dot(x_ref[...], w1_ref[...],
                         preferred_element_type=jnp.float32) * dinv


def _tc_prep(x, w1, degp):
  return pl.pallas_call(
      _prep_body,
      out_shape=[
          jax.ShapeDtypeStruct((NP, NHID), jnp.float32),
          jax.ShapeDtypeStruct((NP, 1), jnp.float32),
      ],
  )(x, w1, degp)


def _layer_body(p_ref, hs_ref, dinv_ref, b_ref, wn_ref, xl_ref, hsn_ref):
  dinv = dinv_ref[...]
  xl = jnp.maximum(dinv * (p_ref[0] + p_ref[1] - hs_ref[...]) + b_ref[...],
                   0.0)
  xl_ref[...] = xl
  hsn_ref[...] = jnp.dot(xl, wn_ref[...],
                         preferred_element_type=jnp.float32) * dinv


def _tc_layer(p, hs, dinv, b, wn):
  return pl.pallas_call(
      _layer_body,
      out_shape=[
          jax.ShapeDtypeStruct((NP, NHID), jnp.float32),
          jax.ShapeDtypeStruct((NP, NHID), jnp.float32),
      ],
  )(p, hs, dinv, b, wn)


def _layer3_body(p_ref, hs_ref, dinv_ref, b_ref, x1_ref, x2_ref, wp_ref,
                 x3_ref, ssw_ref):
  dinv = dinv_ref[...]
  x3 = jnp.maximum(dinv * (p_ref[0] + p_ref[1] - hs_ref[...]) + b_ref[...],
                   0.0)
  x3_ref[...] = x3
  wp = wp_ref[...]
  s = (jnp.dot(x1_ref[...], wp[0:128], preferred_element_type=jnp.float32)
       + jnp.dot(x2_ref[...], wp[128:256], preferred_element_type=jnp.float32)
       + jnp.dot(x3, wp[256:384], preferred_element_type=jnp.float32))
  ssw_ref[...] = s * dinv


def _tc_layer3(p, hs, dinv, b, x1, x2, wp):
  return pl.pallas_call(
      _layer3_body,
      out_shape=[
          jax.ShapeDtypeStruct((NP, NHID), jnp.float32),
          jax.ShapeDtypeStruct((NP, 1), jnp.float32),
      ],
  )(p, hs, dinv, b, x1, x2, wp)


def _poolA_body(sp_ref, ssw_ref, dinv_ref, bp_ref, batch_ref,
                wk_ref, keep_ref, kcol_ref):
  score = (dinv_ref[...]
           * (sp_ref[0] + sp_ref[1] - ssw_ref[...])
           + bp_ref[0, 0])
  batch = batch_ref[...]
  gids = lax.broadcasted_iota(jnp.int32, (1, G), 1)
  oh = (batch == gids)
  counts = jnp.sum(oh.astype(jnp.float32), axis=0, keepdims=True)
  kf = jnp.ceil(0.5 * counts)

  u = lax.bitcast_convert_type(score, jnp.uint32)
  flip = jnp.where(u >= jnp.uint32(0x80000000),
                   jnp.uint32(0xFFFFFFFF), jnp.uint32(0x80000000))
  key = u ^ flip

  def bs_body(i, t):
    b = (jnp.uint32(31) - i.astype(jnp.uint32))
    cand = t | (jnp.uint32(1) << b)
    ge = (key >= cand) & oh
    cnt = jnp.sum(jnp.where(ge, 1.0, 0.0), axis=0, keepdims=True)
    return jnp.where(cnt >= kf, cand, t)

  t = lax.fori_loop(0, 32, bs_body, jnp.zeros((1, G), jnp.uint32))
  ti = lax.bitcast_convert_type(t, jnp.int32)
  tpn = lax.bitcast_convert_type(
      jnp.sum(jnp.where(oh, ti, 0), axis=1, keepdims=True), jnp.uint32)
  keep = key >= tpn
  w = jnp.tanh(score)
  wk_ref[...] = jnp.where(keep, w, 0.0)
  keep_ref[...] = jnp.where(keep, 1.0, 0.0)
  kcol_ref[...] = kf.reshape(G, 1)


def _tc_poolA(sp, ssw, dinv, bp, batch2):
  return pl.pallas_call(
      _poolA_body,
      out_shape=[
          jax.ShapeDtypeStruct((NP, 1), jnp.float32),
          jax.ShapeDtypeStruct((NP, 1), jnp.float32),
          jax.ShapeDtypeStruct((G, 1), jnp.float32),
      ],
  )(sp, ssw, dinv, bp, batch2)


def _poolB_body(x1_ref, x2_ref, x3_ref, wk_ref, keep_ref, kcol_ref,
                batch_ref, batchr_ref, wl1_ref, bl1_ref, wl2_ref, bl2_ref,
                wl3_ref, bl3_ref, out_ref, mx1, mx2, mx3):
  wk = wk_ref[...]
  v1 = x1_ref[...] * wk
  v2 = x2_ref[...] * wk
  v3 = x3_ref[...] * wk
  ohT = (lax.broadcasted_iota(jnp.int32, (G, 1), 0)
         == batchr_ref[...]).astype(jnp.float32)
  ssum1 = jnp.dot(ohT, v1, preferred_element_type=jnp.float32)
  ssum2 = jnp.dot(ohT, v2, preferred_element_type=jnp.float32)
  ssum3 = jnp.dot(ohT, v3, preferred_element_type=jnp.float32)

  neg = jnp.float32(-jnp.inf)
  batch = batch_ref[...]
  keep = keep_ref[...] > 0.0

  def mx_body(g, carry):
    m = (batch == g) & keep
    m1 = jnp.max(jnp.where(m, v1, neg), axis=0, keepdims=True)
    m2 = jnp.max(jnp.where(m, v2, neg), axis=0, keepdims=True)
    m3 = jnp.max(jnp.where(m, v3, neg), axis=0, keepdims=True)
    mx1[pl.ds(g, 1), :] = m1
    mx2[pl.ds(g, 1), :] = m2
    mx3[pl.ds(g, 1), :] = m3
    return carry

  lax.fori_loop(0, G, mx_body, 0)

  kcol = kcol_ref[...]
  inv_k = 1.0 / jnp.maximum(kcol, 1.0)
  gap1, gap2, gap3 = ssum1 * inv_k, ssum2 * inv_k, ssum3 * inv_k
  nz = kcol > 0
  gmp1 = jnp.where(nz, mx1[...], 0.0)
  gmp2 = jnp.where(nz, mx2[...], 0.0)
  gmp3 = jnp.where(nz, mx3[...], 0.0)

  wl1 = wl1_ref[...]
  z = (jnp.dot(gmp1, wl1[0:128], preferred_element_type=jnp.float32)
       + jnp.dot(gmp2, wl1[128:256], preferred_element_type=jnp.float32)
       + jnp.dot(gmp3, wl1[256:384], preferred_element_type=jnp.float32)
       + jnp.dot(gap1, wl1[384:512], preferred_element_type=jnp.float32)
       + jnp.dot(gap2, wl1[512:640], preferred_element_type=jnp.float32)
       + jnp.dot(gap3, wl1[640:768], preferred_element_type=jnp.float32)
       + bl1_ref[...])
  z = jnp.maximum(z, 0.0)
  z = jnp.maximum(jnp.dot(z, wl2_ref[...],
                          preferred_element_type=jnp.float32) + bl2_ref[...],
                  0.0)
  z = jnp.dot(z, wl3_ref[...], preferred_element_type=jnp.float32) + bl3_ref[...]
  m = jnp.max(z, axis=1, keepdims=True)
  lse = jnp.log(jnp.sum(jnp.exp(z - m), axis=1, keepdims=True))
  out_ref[...] = z - m - lse


def _tc_poolB(x1, x2, x3, wk, keepf, kcol, batch2, batchr, wl1, bl1, wl2,
              bl2, wl3, bl3):
  return pl.pallas_call(
      _poolB_body,
      out_shape=jax.ShapeDtypeStruct((G, 10), jnp.float32),
      scratch_shapes=[
          pltpu.VMEM((G, NHID), jnp.float32),
          pltpu.VMEM((G, NHID), jnp.float32),
          pltpu.VMEM((G, NHID), jnp.float32),
      ],
  )(x1, x2, x3, wk, keepf, kcol, batch2, batchr, wl1, bl1, wl2, bl2,
    wl3, bl3)


@jax.jit
def kernel(x, edge_index, batch, W1, b1, W2, b2, W3, b3, Wp, bp,
           Wl1, bl1, Wl2, bl2, Wl3, bl3):
  xp = jnp.pad(x, ((0, NP - N), (0, 0)))
  row = jnp.pad(edge_index[0], (0, EPAD - E), constant_values=DUMMY)
  col = jnp.pad(edge_index[1], (0, EPAD - E), constant_values=DUMMY)
  rowi = row.reshape(NW, NCH, C)
  coli = col.reshape(NW, NCH, C)
  eidx = jnp.concatenate([row.reshape(NW, NCH, 1, C),
                          col.reshape(NW, NCH, 1, C)], axis=2)
  batch_p = jnp.pad(batch, (0, NP - N), constant_values=G)
  batch2 = batch_p.reshape(NP, 1)
  batchr = batch_p.reshape(1, NP)
  ones1 = jnp.ones((NP,), jnp.float32)

  degp = _sc_agg1(ones1, rowi, coli).reshape(2, NP, 1)
  hs1, dinv = _tc_prep(xp, W1, degp)
  p1 = _sc_agg128(hs1, eidx)
  x1, hs2 = _tc_layer(p1, hs1, dinv, b1.reshape(1, NHID), W2)
  p2 = _sc_agg128(hs2, eidx)
  x2, hs3 = _tc_layer(p2, hs2, dinv, b2.reshape(1, NHID), W3)
  p3 = _sc_agg128(hs3, eidx)
  x3, ssw = _tc_layer3(p3, hs3, dinv, b3.reshape(1, NHID), x1, x2, Wp)
  sp = _sc_agg1(ssw.reshape(NP), rowi, coli).reshape(2, NP, 1)
  wk, keepf, kcol = _tc_poolA(sp, ssw, dinv, bp.reshape(1, 1), batch2)
  return _tc_poolB(x1, x2, x3, wk, keepf, kcol, batch2, batchr, Wl1,
                   bl1.reshape(1, 2 * NHID), Wl2, bl2.reshape(1, NHID),
                   Wl3, bl3.reshape(1, 10))
